# Initial kernel scaffold; baseline (speedup 1.0000x reference)
import jax
import jax.numpy as jnp
from jax.experimental import pallas as pl


def kernel(target, context, negative_samples, embeddings):
    raise NotImplementedError("write your pallas kernel here")


# speedup vs baseline: 1.6264x; 1.6264x over previous
import functools

import jax
import jax.numpy as jnp
from jax import lax
from jax.experimental import pallas as pl
from jax.experimental.pallas import tpu as pltpu
from jax.experimental.pallas import tpu_sc as plsc

_D = 128
_CHUNK = 128
_LANES = 16
_GROUPS = _D // _LANES


@functools.lru_cache(maxsize=None)
def _make_partial_kernel(nw: int, nc: int, chunks: int):
  mesh = plsc.VectorSubcoreMesh(core_axis_name="c", subcore_axis_name="s")

  @functools.partial(
      pl.kernel,
      mesh=mesh,
      out_type=jax.ShapeDtypeStruct((nw, _LANES), jnp.float32),
      scratch_types=[
          pltpu.VMEM((chunks, _CHUNK), jnp.int32),
          pltpu.VMEM((chunks, _CHUNK), jnp.int32),
          pltpu.VMEM((chunks, _CHUNK), jnp.int32),
          pltpu.VMEM((_CHUNK, _D), jnp.float32),
          pltpu.VMEM((_CHUNK, _D), jnp.float32),
          pltpu.VMEM((_CHUNK, _D), jnp.float32),
          pltpu.VMEM((_LANES,), jnp.float32),
          pltpu.SemaphoreType.DMA,
          pltpu.SemaphoreType.DMA,
          pltpu.SemaphoreType.DMA,
      ],
  )
  def partial_kernel(t_hbm, c_hbm, n_hbm, emb_hbm, out_hbm,
                     tidx, cidx, nidx, trows, crows, nrows, accv,
                     sem_t, sem_c, sem_n):
    wid = lax.axis_index("s") * nc + lax.axis_index("c")
    pltpu.sync_copy(t_hbm.at[wid], tidx)
    pltpu.sync_copy(c_hbm.at[wid], cidx)
    pltpu.sync_copy(n_hbm.at[wid], nidx)

    acc = tuple(jnp.zeros((_LANES,), jnp.float32) for _ in range(_GROUPS))
    for j in range(chunks):
      cp_t = pltpu.async_copy(emb_hbm.at[tidx.at[j]], trows, sem_t)
      cp_c = pltpu.async_copy(emb_hbm.at[cidx.at[j]], crows, sem_c)
      cp_n = pltpu.async_copy(emb_hbm.at[nidx.at[j]], nrows, sem_n)
      cp_t.wait()
      cp_c.wait()
      cp_n.wait()

      def row_body(i, a):
        new = []
        for g in range(_GROUPS):
          sl = pl.ds(g * _LANES, _LANES)
          new.append(a[g] + trows[i, sl] * (crows[i, sl] - nrows[i, sl]))
        return tuple(new)

      acc = lax.fori_loop(0, _CHUNK, row_body, acc)

    total = acc[0]
    for g in range(1, _GROUPS):
      total = total + acc[g]
    accv[...] = total
    pltpu.sync_copy(accv, out_hbm.at[wid])

  return partial_kernel


def kernel(target, context, negative_samples, embeddings):
  b = target.shape[0]
  info = plsc.get_sparse_core_info()
  nw = info.num_cores * info.num_subcores
  per_w = b // nw
  chunks = per_w // _CHUNK
  t = target.reshape(nw, chunks, _CHUNK)
  c = context.reshape(nw, chunks, _CHUNK)
  n = negative_samples.reshape(nw, chunks, _CHUNK)
  partials = _make_partial_kernel(nw, info.num_cores, chunks)(
      t, c, n, embeddings)
  return -(jnp.sum(partials) / b)


# speedup vs baseline: 1.8244x; 1.1217x over previous
import functools

import jax
import jax.numpy as jnp
from jax import lax
from jax.experimental import pallas as pl
from jax.experimental.pallas import tpu as pltpu
from jax.experimental.pallas import tpu_sc as plsc

_D = 128
_CHUNK = 128
_LANES = 16
_GROUPS = _D // _LANES


@functools.lru_cache(maxsize=None)
def _make_partial_kernel(nw: int, nc: int, chunks: int):
  mesh = plsc.VectorSubcoreMesh(core_axis_name="c", subcore_axis_name="s")

  @functools.partial(
      pl.kernel,
      mesh=mesh,
      out_type=jax.ShapeDtypeStruct((nw, _LANES), jnp.float32),
      scratch_types=[
          pltpu.VMEM((chunks, _CHUNK), jnp.int32),
          pltpu.VMEM((chunks, _CHUNK), jnp.int32),
          pltpu.VMEM((chunks, _CHUNK), jnp.int32),
          pltpu.VMEM((_CHUNK, _D), jnp.float32),
          pltpu.VMEM((_CHUNK, _D), jnp.float32),
          pltpu.VMEM((_CHUNK, _D), jnp.float32),
          pltpu.VMEM((_CHUNK, _D), jnp.float32),
          pltpu.VMEM((_CHUNK, _D), jnp.float32),
          pltpu.VMEM((_CHUNK, _D), jnp.float32),
          pltpu.VMEM((_LANES,), jnp.float32),
          pltpu.SemaphoreType.DMA,
          pltpu.SemaphoreType.DMA,
      ],
  )
  def partial_kernel(t_hbm, c_hbm, n_hbm, emb_hbm, out_hbm,
                     tidx, cidx, nidx, tr0, cr0, nr0, tr1, cr1, nr1, accv,
                     sem0, sem1):
    wid = lax.axis_index("s") * nc + lax.axis_index("c")
    pltpu.sync_copy(t_hbm.at[wid], tidx)
    pltpu.sync_copy(c_hbm.at[wid], cidx)
    pltpu.sync_copy(n_hbm.at[wid], nidx)

    bufs = ((tr0, cr0, nr0, sem0), (tr1, cr1, nr1, sem1))

    def fire(j, buf):
      tr, cr, nr, sem = buf
      return (pltpu.async_copy(emb_hbm.at[tidx.at[j]], tr, sem),
              pltpu.async_copy(emb_hbm.at[cidx.at[j]], cr, sem),
              pltpu.async_copy(emb_hbm.at[nidx.at[j]], nr, sem))

    acc = tuple(jnp.zeros((_LANES,), jnp.float32) for _ in range(2 * _GROUPS))
    cps = fire(0, bufs[0])
    for j in range(chunks):
      for cp in cps:
        cp.wait()
      if j + 1 < chunks:
        cps = fire(j + 1, bufs[(j + 1) % 2])
      tr, cr, nr, _ = bufs[j % 2]

      def row_body(i, a, tr=tr, cr=cr, nr=nr):
        r0 = 2 * i
        r1 = r0 + 1
        new = list(a)
        for g in range(_GROUPS):
          sl = pl.ds(g * _LANES, _LANES)
          new[g] = new[g] + tr[r0, sl] * (cr[r0, sl] - nr[r0, sl])
          new[_GROUPS + g] = (
              new[_GROUPS + g] + tr[r1, sl] * (cr[r1, sl] - nr[r1, sl]))
        return tuple(new)

      acc = lax.fori_loop(0, _CHUNK // 2, row_body, acc)

    total = acc[0]
    for g in range(1, 2 * _GROUPS):
      total = total + acc[g]
    accv[...] = total
    pltpu.sync_copy(accv, out_hbm.at[wid])

  return partial_kernel


def kernel(target, context, negative_samples, embeddings):
  b = target.shape[0]
  info = plsc.get_sparse_core_info()
  nw = info.num_cores * info.num_subcores
  per_w = b // nw
  chunks = per_w // _CHUNK
  t = target.reshape(nw, chunks, _CHUNK)
  c = context.reshape(nw, chunks, _CHUNK)
  n = negative_samples.reshape(nw, chunks, _CHUNK)
  partials = _make_partial_kernel(nw, info.num_cores, chunks)(
      t, c, n, embeddings)
  return -(jnp.sum(partials) / b)
